# baseline (speedup 1.0000x reference)
import jax
import jax.numpy as jnp
from jax.experimental import pallas as pl
import jax.experimental.pallas.tpu as pltpu
from jax.experimental import layout as _layout

E = 64
TOP_K = 2
D_MODEL = 128
BATCH = 64
SEQ_LEN = 4096

CHUNK_B = 4
N_CH = BATCH // CHUNK_B
RING = 4


def _router_kernel(x_hbm, c_ref, scores_out_ref, idx_out_ref, acc_ref, *rest):
    bufs = rest[:RING]
    sems = rest[RING:]
    H = CHUNK_B // 2

    def copies(k):
        b = bufs[k % RING]
        s0 = sems[2 * (k % RING)]
        s1 = sems[2 * (k % RING) + 1]
        return (
            pltpu.make_async_copy(x_hbm.at[pl.ds(k * CHUNK_B, H)], b.at[0:H], s0),
            pltpu.make_async_copy(x_hbm.at[pl.ds(k * CHUNK_B + H, H)], b.at[H:CHUNK_B], s1),
        )

    for k in range(RING):
        for c in copies(k):
            c.start()
    for k in range(N_CH):
        for c in copies(k):
            c.wait()
        acc_ref[pl.ds(k * CHUNK_B, CHUNK_B), :] = jnp.sum(bufs[k % RING][...], axis=1)
        if k + RING < N_CH:
            for c in copies(k + RING):
                c.start()

    c = c_ref[...]
    cn = jnp.sqrt(jnp.sum(c * c, axis=1, keepdims=True))
    c = c / jnp.maximum(cn, 1e-12)

    q = acc_ref[...] * (1.0 / SEQ_LEN)
    qn = jnp.sqrt(jnp.sum(q * q, axis=1, keepdims=True))
    q = q / jnp.maximum(qn, 1e-12)

    scores = jax.lax.dot_general(
        q, c, (((1,), (1,)), ((), ())), preferred_element_type=jnp.float32
    )

    iota = jax.lax.broadcasted_iota(jnp.int32, (BATCH, E), 1)
    m1 = jnp.max(scores, axis=1, keepdims=True)
    i1 = jnp.min(
        jnp.where(scores == m1, iota, jnp.int32(2**30)), axis=1, keepdims=True
    )
    masked = jnp.where(iota == i1, -jnp.inf, scores)
    m2 = jnp.max(masked, axis=1, keepdims=True)
    i2 = jnp.min(
        jnp.where(masked == m2, iota, jnp.int32(2**30)), axis=1, keepdims=True
    )

    scores_out_ref[:, 0:1] = m1
    scores_out_ref[:, 1:2] = m2
    idx_out_ref[:, 0:1] = i1
    idx_out_ref[:, 1:2] = i2


@jax.jit
def kernel(x, centroids):
    top_scores, top_idx = pl.pallas_call(
        _router_kernel,
        in_specs=[
            pl.BlockSpec(memory_space=pl.ANY),
            pl.BlockSpec(memory_space=pltpu.MemorySpace.VMEM),
        ],
        out_specs=[
            pl.BlockSpec(memory_space=pltpu.MemorySpace.VMEM),
            pl.BlockSpec(memory_space=pltpu.MemorySpace.VMEM),
        ],
        out_shape=[
            jax.ShapeDtypeStruct((BATCH, TOP_K), jnp.float32),
            jax.ShapeDtypeStruct((BATCH, TOP_K), jnp.int32),
        ],
        scratch_shapes=(
            [pltpu.VMEM((BATCH, D_MODEL), jnp.float32)]
            + [pltpu.VMEM((CHUNK_B, SEQ_LEN, D_MODEL), jnp.float32) for _ in range(RING)]
            + [pltpu.SemaphoreType.DMA for _ in range(2 * RING)]
        ),
    )(x, centroids)
    return top_scores, top_idx


# speedup vs baseline: 1.0971x; 1.0971x over previous
import jax
import jax.numpy as jnp
from jax.experimental import pallas as pl
import jax.experimental.pallas.tpu as pltpu

E = 64
TOP_K = 2
D_MODEL = 128
BATCH = 64
SEQ_LEN = 4096

B_BLK = 4
N_BBLKS = BATCH // B_BLK


def _router_kernel(x_ref, c_ref, scores_out_ref, idx_out_ref, acc_ref):
    i = pl.program_id(0)
    acc_ref[i] = jnp.sum(x_ref[...], axis=1)

    @pl.when(i == N_BBLKS - 1)
    def _finalize():
        c = c_ref[...]
        cn = jnp.sqrt(jnp.sum(c * c, axis=1, keepdims=True))
        c = c / jnp.maximum(cn, 1e-12)

        q = acc_ref[...].reshape(BATCH, D_MODEL) * (1.0 / SEQ_LEN)
        qn = jnp.sqrt(jnp.sum(q * q, axis=1, keepdims=True))
        q = q / jnp.maximum(qn, 1e-12)

        scores = jax.lax.dot_general(
            q, c, (((1,), (1,)), ((), ())), preferred_element_type=jnp.float32
        )

        iota = jax.lax.broadcasted_iota(jnp.int32, (BATCH, E), 1)
        m1 = jnp.max(scores, axis=1, keepdims=True)
        i1 = jnp.min(
            jnp.where(scores == m1, iota, jnp.int32(2**30)), axis=1, keepdims=True
        )
        masked = jnp.where(iota == i1, -jnp.inf, scores)
        m2 = jnp.max(masked, axis=1, keepdims=True)
        i2 = jnp.min(
            jnp.where(masked == m2, iota, jnp.int32(2**30)), axis=1, keepdims=True
        )

        scores_out_ref[:, 0:1] = m1
        scores_out_ref[:, 1:2] = m2
        idx_out_ref[:, 0:1] = i1
        idx_out_ref[:, 1:2] = i2


def _acc_index_map(i):
    return (i, 0)


@jax.jit
def kernel(x, centroids):
    top_scores, top_idx = pl.pallas_call(
        _router_kernel,
        grid=(N_BBLKS,),
        in_specs=[
            pl.BlockSpec((B_BLK, SEQ_LEN, D_MODEL), lambda i: (i, 0, 0)),
            pl.BlockSpec((E, D_MODEL), lambda i: (0, 0)),
        ],
        out_specs=[
            pl.BlockSpec((BATCH, TOP_K), lambda i: (0, 0)),
            pl.BlockSpec((BATCH, TOP_K), lambda i: (0, 0)),
        ],
        out_shape=[
            jax.ShapeDtypeStruct((BATCH, TOP_K), jnp.float32),
            jax.ShapeDtypeStruct((BATCH, TOP_K), jnp.int32),
        ],
        scratch_shapes=[pltpu.VMEM((N_BBLKS, B_BLK, D_MODEL), jnp.float32)],
        compiler_params=pltpu.CompilerParams(
            dimension_semantics=("arbitrary",),
        ),
    )(x, centroids)
    return top_scores, top_idx
